# baseline (speedup 1.0000x reference)
import functools

import jax
import jax.numpy as jnp
from jax import lax
from jax.experimental import pallas as pl
from jax.experimental.pallas import tpu as pltpu
from jax.experimental.pallas import tpu_sc as plsc

_DIM = 256
_K = 1024
_TOK = 1024
_UNROLL = 2
_NSTEP = 16 // _UNROLL
_NTOTAL = 16 * _TOK
_NUMEL = _NTOTAL * _DIM

_NC = 2
_NS = 16
_NW = _NC * _NS
_CHUNK = _NTOTAL // _NW
_L = 16


def _vq_body(x_ref, cb_ref, q_ref, idx_ref, loss_ref, sse_ref):
    s = pl.program_id(0)

    @pl.when(s == 0)
    def _init():
        sse_ref[0] = jnp.float32(0.0)

    C = cb_ref[...]
    cn = jnp.sum(C * C, axis=1)
    row = jax.lax.broadcasted_iota(jnp.int32, (_K, _TOK), 0)

    for u in range(_UNROLL):
        X = x_ref[u]

        mm = jax.lax.dot_general(C, X, (((1,), (0,)), ((), ())),
                                 preferred_element_type=jnp.float32)
        rn = jnp.sum(X * X, axis=0)
        d = (rn[None, :] - 2.0 * mm) + cn[:, None]

        dmin = jnp.min(d, axis=0)
        idx = jnp.min(jnp.where(d == dmin[None, :], row, _K), axis=0)
        idx_ref[u, 0] = idx

        O = (row == idx[None, :]).astype(jnp.float32)
        Q = jax.lax.dot_general(C, O, (((0,), (0,)), ((), ())),
                                preferred_element_type=jnp.float32)
        q_ref[u] = X + (Q - X)

        sse_ref[0] += jnp.sum(dmin)

    @pl.when(s == _NSTEP - 1)
    def _fin():
        loss_ref[...] = jnp.full((1, 1), sse_ref[0] * jnp.float32(1.25 / _NUMEL),
                                 jnp.float32)


def _sc_bincount_body(idx_hbm, out_hbm, idx_v, counts_v):
    wid = lax.axis_index("s") * _NC + lax.axis_index("c")
    s = wid // 2
    half = wid % 2

    zeros16 = jnp.zeros((_L,), jnp.float32)
    for grp in range(_L * _K // _L):
        counts_v[pl.ds(grp * _L, _L)] = zeros16

    pltpu.sync_copy(idx_hbm.at[s, 0, pl.ds(half * _CHUNK, _CHUNK)], idx_v)

    lane_base = lax.iota(jnp.int32, _L) * _K
    ones16 = jnp.ones((_L,), jnp.float32)
    for j in range(_CHUNK // _L):
        iv = idx_v[pl.ds(j * _L, _L)]
        plsc.addupdate_scatter(counts_v, [lane_base + iv], ones16)

    for r in range(_L):
        pltpu.sync_copy(counts_v.at[pl.ds(r * _K, _K)], out_hbm.at[wid, r])


def _entropy_body(p_ref, ent_ref):
    counts = jnp.sum(p_ref[...], axis=0)
    total = jnp.maximum(jnp.sum(counts), 1.0)
    probs = counts / total
    safe = jnp.maximum(probs, 1e-30)
    ent = -jnp.sum(jnp.where(probs > 0,
                             probs * (jnp.log(safe) / jnp.log(2.0)),
                             0.0))
    ent_ref[...] = jnp.full((1, 1), ent, jnp.float32)


@jax.jit
def kernel(x, codebook):
    xr = x.reshape(16, _DIM, _TOK)
    q, idx, loss = pl.pallas_call(
        _vq_body,
        grid=(_NSTEP,),
        in_specs=[
            pl.BlockSpec((_UNROLL, _DIM, _TOK), lambda s: (s, 0, 0)),
            pl.BlockSpec((_K, _DIM), lambda s: (0, 0)),
        ],
        out_specs=[
            pl.BlockSpec((_UNROLL, _DIM, _TOK), lambda s: (s, 0, 0)),
            pl.BlockSpec((_UNROLL, 1, _TOK), lambda s: (s, 0, 0)),
            pl.BlockSpec((1, 1), lambda s: (0, 0)),
        ],
        out_shape=[
            jax.ShapeDtypeStruct((16, _DIM, _TOK), jnp.float32),
            jax.ShapeDtypeStruct((16, 1, _TOK), jnp.int32),
            jax.ShapeDtypeStruct((1, 1), jnp.float32),
        ],
        scratch_shapes=[
            pltpu.SMEM((1,), jnp.float32),
        ],
    )(xr, codebook)

    mesh = plsc.VectorSubcoreMesh(core_axis_name="c", subcore_axis_name="s")
    sc_bincount = functools.partial(
        pl.kernel, mesh=mesh,
        out_type=jax.ShapeDtypeStruct((_NW, _L, _K), jnp.float32),
        scratch_types=[
            pltpu.VMEM((_CHUNK,), jnp.int32),
            pltpu.VMEM((_L * _K,), jnp.float32),
        ],
        compiler_params=pltpu.CompilerParams(needs_layout_passes=False),
    )(_sc_bincount_body)
    partials = sc_bincount(idx)

    ent = pl.pallas_call(
        _entropy_body,
        out_shape=jax.ShapeDtypeStruct((1, 1), jnp.float32),
    )(partials.reshape(_NW * _L, _K))

    quantized_st = q.reshape(x.shape)
    indices = idx.reshape(_NTOTAL)
    return quantized_st, indices, loss[0, 0], ent[0, 0]


# speedup vs baseline: 1.0005x; 1.0005x over previous
import functools

import jax
import jax.numpy as jnp
from jax import lax
from jax.experimental import pallas as pl
from jax.experimental.pallas import tpu as pltpu
from jax.experimental.pallas import tpu_sc as plsc

_DIM = 256
_K = 1024
_TOK = 1024
_UNROLL = 4
_NSTEP = 16 // _UNROLL
_NTOTAL = 16 * _TOK
_NUMEL = _NTOTAL * _DIM

_NC = 2
_NS = 16
_NW = _NC * _NS
_CHUNK = _NTOTAL // _NW
_L = 16


def _vq_body(x_ref, cb_ref, q_ref, idx_ref, loss_ref, sse_ref):
    s = pl.program_id(0)

    @pl.when(s == 0)
    def _init():
        sse_ref[0] = jnp.float32(0.0)

    C = cb_ref[...]
    cn = jnp.sum(C * C, axis=1)
    row = jax.lax.broadcasted_iota(jnp.int32, (_K, _TOK), 0)

    for u in range(_UNROLL):
        X = x_ref[u]

        mm = jax.lax.dot_general(C, X, (((1,), (0,)), ((), ())),
                                 preferred_element_type=jnp.float32)
        rn = jnp.sum(X * X, axis=0)
        d = (rn[None, :] - 2.0 * mm) + cn[:, None]

        dmin = jnp.min(d, axis=0)
        idx = jnp.min(jnp.where(d == dmin[None, :], row, _K), axis=0)
        idx_ref[u, 0] = idx

        O = (row == idx[None, :]).astype(jnp.float32)
        Q = jax.lax.dot_general(C, O, (((0,), (0,)), ((), ())),
                                preferred_element_type=jnp.float32)
        q_ref[u] = X + (Q - X)

        sse_ref[0] += jnp.sum(dmin)

    @pl.when(s == _NSTEP - 1)
    def _fin():
        loss_ref[...] = jnp.full((1, 1), sse_ref[0] * jnp.float32(1.25 / _NUMEL),
                                 jnp.float32)


def _sc_bincount_body(idx_hbm, out_hbm, idx_v, counts_v):
    wid = lax.axis_index("s") * _NC + lax.axis_index("c")
    s = wid // 2
    half = wid % 2

    zeros16 = jnp.zeros((_L,), jnp.float32)
    for grp in range(_L * _K // _L):
        counts_v[pl.ds(grp * _L, _L)] = zeros16

    pltpu.sync_copy(idx_hbm.at[s, 0, pl.ds(half * _CHUNK, _CHUNK)], idx_v)

    lane_base = lax.iota(jnp.int32, _L) * _K
    ones16 = jnp.ones((_L,), jnp.float32)
    for j in range(_CHUNK // _L):
        iv = idx_v[pl.ds(j * _L, _L)]
        plsc.addupdate_scatter(counts_v, [lane_base + iv], ones16)

    for r in range(_L):
        pltpu.sync_copy(counts_v.at[pl.ds(r * _K, _K)], out_hbm.at[wid, r])


def _entropy_body(p_ref, ent_ref):
    counts = jnp.sum(p_ref[...], axis=0)
    total = jnp.maximum(jnp.sum(counts), 1.0)
    probs = counts / total
    safe = jnp.maximum(probs, 1e-30)
    ent = -jnp.sum(jnp.where(probs > 0,
                             probs * (jnp.log(safe) / jnp.log(2.0)),
                             0.0))
    ent_ref[...] = jnp.full((1, 1), ent, jnp.float32)


@jax.jit
def kernel(x, codebook):
    xr = x.reshape(16, _DIM, _TOK)
    q, idx, loss = pl.pallas_call(
        _vq_body,
        grid=(_NSTEP,),
        in_specs=[
            pl.BlockSpec((_UNROLL, _DIM, _TOK), lambda s: (s, 0, 0)),
            pl.BlockSpec((_K, _DIM), lambda s: (0, 0)),
        ],
        out_specs=[
            pl.BlockSpec((_UNROLL, _DIM, _TOK), lambda s: (s, 0, 0)),
            pl.BlockSpec((_UNROLL, 1, _TOK), lambda s: (s, 0, 0)),
            pl.BlockSpec((1, 1), lambda s: (0, 0)),
        ],
        out_shape=[
            jax.ShapeDtypeStruct((16, _DIM, _TOK), jnp.float32),
            jax.ShapeDtypeStruct((16, 1, _TOK), jnp.int32),
            jax.ShapeDtypeStruct((1, 1), jnp.float32),
        ],
        scratch_shapes=[
            pltpu.SMEM((1,), jnp.float32),
        ],
        compiler_params=pltpu.CompilerParams(
            vmem_limit_bytes=128 * 1024 * 1024),
    )(xr, codebook)

    mesh = plsc.VectorSubcoreMesh(core_axis_name="c", subcore_axis_name="s")
    sc_bincount = functools.partial(
        pl.kernel, mesh=mesh,
        out_type=jax.ShapeDtypeStruct((_NW, _L, _K), jnp.float32),
        scratch_types=[
            pltpu.VMEM((_CHUNK,), jnp.int32),
            pltpu.VMEM((_L * _K,), jnp.float32),
        ],
        compiler_params=pltpu.CompilerParams(needs_layout_passes=False),
    )(_sc_bincount_body)
    partials = sc_bincount(idx)

    ent = pl.pallas_call(
        _entropy_body,
        out_shape=jax.ShapeDtypeStruct((1, 1), jnp.float32),
    )(partials.reshape(_NW * _L, _K))

    quantized_st = q.reshape(x.shape)
    indices = idx.reshape(_NTOTAL)
    return quantized_st, indices, loss[0, 0], ent[0, 0]


# speedup vs baseline: 1.0913x; 1.0908x over previous
import jax
import jax.numpy as jnp
from jax.experimental import pallas as pl
from jax.experimental.pallas import tpu as pltpu

_DIM = 256
_K = 1024
_TOK = 1024
_UNROLL = 4
_NSTEP = 16 // _UNROLL
_NTOTAL = 16 * _TOK
_NUMEL = _NTOTAL * _DIM


def _vq_body(x_ref, cb_ref, q_ref, idx_ref, loss_ref, ent_ref,
             counts_ref, sse_ref):
    s = pl.program_id(0)

    @pl.when(s == 0)
    def _init():
        counts_ref[...] = jnp.zeros_like(counts_ref)
        sse_ref[0] = jnp.float32(0.0)

    C = cb_ref[...]
    cn = jnp.sum(C * C, axis=1)
    row = jax.lax.broadcasted_iota(jnp.int32, (_K, _TOK), 0)
    ones_t = jnp.ones((_TOK, 1), jnp.bfloat16)

    for u in range(_UNROLL):
        X = x_ref[u]

        mm = jax.lax.dot_general(C, X, (((1,), (0,)), ((), ())),
                                 preferred_element_type=jnp.float32)
        rn = jnp.sum(X * X, axis=0)
        d = (rn[None, :] - 2.0 * mm) + cn[:, None]

        dmin = jnp.min(d, axis=0)
        idx = jnp.min(jnp.where(d == dmin[None, :], row, _K), axis=0)
        idx_ref[u, 0] = idx

        O = (row == idx[None, :]).astype(jnp.float32)
        Q = jax.lax.dot_general(C, O, (((0,), (0,)), ((), ())),
                                preferred_element_type=jnp.float32)
        q_ref[u] = X + (Q - X)

        counts_ref[...] += jax.lax.dot_general(
            O.astype(jnp.bfloat16), ones_t, (((1,), (0,)), ((), ())),
            preferred_element_type=jnp.float32)
        sse_ref[0] += jnp.sum(dmin)

    @pl.when(s == _NSTEP - 1)
    def _fin():
        loss_ref[...] = jnp.full((1, 1), sse_ref[0] * jnp.float32(1.25 / _NUMEL),
                                 jnp.float32)
        counts = counts_ref[:, 0]
        total = jnp.maximum(jnp.sum(counts), 1.0)
        probs = counts / total
        safe = jnp.maximum(probs, 1e-30)
        ent = -jnp.sum(jnp.where(probs > 0,
                                 probs * (jnp.log(safe) / jnp.log(2.0)),
                                 0.0))
        ent_ref[...] = jnp.full((1, 1), ent, jnp.float32)


@jax.jit
def kernel(x, codebook):
    xr = x.reshape(16, _DIM, _TOK)
    q, idx, loss, ent = pl.pallas_call(
        _vq_body,
        grid=(_NSTEP,),
        in_specs=[
            pl.BlockSpec((_UNROLL, _DIM, _TOK), lambda s: (s, 0, 0)),
            pl.BlockSpec((_K, _DIM), lambda s: (0, 0)),
        ],
        out_specs=[
            pl.BlockSpec((_UNROLL, _DIM, _TOK), lambda s: (s, 0, 0)),
            pl.BlockSpec((_UNROLL, 1, _TOK), lambda s: (s, 0, 0)),
            pl.BlockSpec((1, 1), lambda s: (0, 0)),
            pl.BlockSpec((1, 1), lambda s: (0, 0)),
        ],
        out_shape=[
            jax.ShapeDtypeStruct((16, _DIM, _TOK), jnp.float32),
            jax.ShapeDtypeStruct((16, 1, _TOK), jnp.int32),
            jax.ShapeDtypeStruct((1, 1), jnp.float32),
            jax.ShapeDtypeStruct((1, 1), jnp.float32),
        ],
        scratch_shapes=[
            pltpu.VMEM((_K, 1), jnp.float32),
            pltpu.SMEM((1,), jnp.float32),
        ],
    )(xr, codebook)
    quantized_st = q.reshape(x.shape)
    indices = idx.reshape(_NTOTAL)
    return quantized_st, indices, loss[0, 0], ent[0, 0]


# speedup vs baseline: 1.1033x; 1.0111x over previous
import functools

import jax
import jax.numpy as jnp
from jax.experimental import pallas as pl
from jax.experimental.pallas import tpu as pltpu

_DIM = 256
_K = 1024
_TOK = 1024
_UNROLL = 4
_NSTEP = 16 // _UNROLL
_NTOTAL = 16 * _TOK
_NUMEL = _NTOTAL * _DIM


def _vq_body(x_ref, cb_ref, q_ref, idx_ref, loss_ref, ent_ref,
             counts_ref, sse_ref):
    s = pl.program_id(0)

    @pl.when(s == 0)
    def _init():
        counts_ref[...] = jnp.zeros_like(counts_ref)
        sse_ref[0] = jnp.float32(0.0)

    C = cb_ref[...]
    cn = jnp.sum(C * C, axis=1)
    row = jax.lax.broadcasted_iota(jnp.int32, (_K, _TOK), 0)
    ones_t = jnp.ones((_TOK, 1), jnp.float32)

    for u in range(_UNROLL):
        X = x_ref[u]

        mm = jax.lax.dot_general(C, X, (((1,), (0,)), ((), ())),
                                 preferred_element_type=jnp.float32)
        rn = jnp.sum(X * X, axis=0)
        d = (rn[None, :] - 2.0 * mm) + cn[:, None]

        dmin = jnp.min(d, axis=0)
        idx = jnp.min(jnp.where(d == dmin[None, :], row, _K), axis=0)
        idx_ref[u, 0] = idx

        O = (row == idx[None, :]).astype(jnp.float32)
        Q = jax.lax.dot_general(C, O, (((0,), (0,)), ((), ())),
                                preferred_element_type=jnp.float32)
        q_ref[u] = X + (Q - X)

        counts_ref[...] += jax.lax.dot_general(
            O, ones_t, (((1,), (0,)), ((), ())),
            preferred_element_type=jnp.float32)
        sse_ref[0] += jnp.sum(dmin)

    @pl.when(s == _NSTEP - 1)
    def _fin():
        loss_ref[...] = jnp.full((1, 1), sse_ref[0] * jnp.float32(1.25 / _NUMEL),
                                 jnp.float32)
        counts = counts_ref[:, 0]
        total = jnp.maximum(jnp.sum(counts), 1.0)
        probs = counts / total
        safe = jnp.maximum(probs, 1e-30)
        ent = -jnp.sum(jnp.where(probs > 0,
                                 probs * (jnp.log(safe) / jnp.log(2.0)),
                                 0.0))
        ent_ref[...] = jnp.full((1, 1), ent, jnp.float32)


@jax.jit
def kernel(x, codebook):
    xr = x.reshape(16, _DIM, _TOK)
    q, idx, loss, ent = pl.pallas_call(
        _vq_body,
        grid=(_NSTEP,),
        in_specs=[
            pl.BlockSpec((_UNROLL, _DIM, _TOK), lambda s: (s, 0, 0)),
            pl.BlockSpec((_K, _DIM), lambda s: (0, 0)),
        ],
        out_specs=[
            pl.BlockSpec((_UNROLL, _DIM, _TOK), lambda s: (s, 0, 0)),
            pl.BlockSpec((_UNROLL, 1, _TOK), lambda s: (s, 0, 0)),
            pl.BlockSpec((1, 1), lambda s: (0, 0)),
            pl.BlockSpec((1, 1), lambda s: (0, 0)),
        ],
        out_shape=[
            jax.ShapeDtypeStruct((16, _DIM, _TOK), jnp.float32),
            jax.ShapeDtypeStruct((16, 1, _TOK), jnp.int32),
            jax.ShapeDtypeStruct((1, 1), jnp.float32),
            jax.ShapeDtypeStruct((1, 1), jnp.float32),
        ],
        scratch_shapes=[
            pltpu.VMEM((_K, 1), jnp.float32),
            pltpu.SMEM((1,), jnp.float32),
        ],
    )(xr, codebook)
    quantized_st = q.reshape(x.shape)
    indices = idx.reshape(_NTOTAL)
    return quantized_st, indices, loss[0, 0], ent[0, 0]
